# baseline (speedup 1.0000x reference)
import functools

import jax
import jax.numpy as jnp
from jax import lax
from jax.experimental import pallas as pl
from jax.experimental.pallas import tpu as pltpu
from jax.experimental.pallas import tpu_sc as plsc

B = 128
V = 100000
K = 1024
NW = 32
ROWS_PER_W = B // NW
CHUNK = 128
NCH = K // CHUNK
VC = 2048
NB = (V + VC - 1) // VC
NEG_INF = float("-inf")


def _sc_gather_body(xflat, ids_hbm, tok_hbm, top_out, xt_out,
                    ids_v, fidx_v, row_v, tok_v, tfidx_v, xtrow_v, sem):
    wid = lax.axis_index("c") * 16 + lax.axis_index("s")
    pltpu.sync_copy(ids_hbm, ids_v)

    def row_body(j, carry):
        r = wid * ROWS_PER_W + j
        base = r * V
        for c in range(NCH):
            for i in range(CHUNK // 16):
                fidx_v[c, pl.ds(i * 16, 16)] = (
                    ids_v[pl.ds(c * CHUNK + i * 16, 16)] + base)
        cps = [pltpu.async_copy(xflat.at[fidx_v.at[c]], row_v.at[c], sem)
               for c in range(NCH)]
        for cp in cps:
            cp.wait()
        pltpu.sync_copy(row_v, top_out.at[r])
        return carry

    lax.fori_loop(0, ROWS_PER_W, row_body, 0)

    @pl.when(wid == 0)
    def _():
        pltpu.sync_copy(tok_hbm, tok_v)
        for i in range(B // 16):
            tfidx_v[pl.ds(i * 16, 16)] = (
                tok_v[pl.ds(i * 16, 16)]
                + (lax.iota(jnp.int32, 16) + i * 16) * V)
        pltpu.async_copy(xflat.at[tfidx_v], xtrow_v, sem).wait()
        pltpu.sync_copy(xtrow_v, xt_out)


_sc_gather = functools.partial(
    pl.kernel,
    mesh=plsc.VectorSubcoreMesh(core_axis_name="c", subcore_axis_name="s"),
    out_type=[
        jax.ShapeDtypeStruct((B, NCH, CHUNK), jnp.float32),
        jax.ShapeDtypeStruct((B,), jnp.float32),
    ],
    scratch_types=[
        pltpu.VMEM((K,), jnp.int32),
        pltpu.VMEM((NCH, CHUNK), jnp.int32),
        pltpu.VMEM((NCH, CHUNK), jnp.float32),
        pltpu.VMEM((B,), jnp.int32),
        pltpu.VMEM((B,), jnp.int32),
        pltpu.VMEM((B,), jnp.float32),
        pltpu.SemaphoreType.DMA,
    ],
)(_sc_gather_body)


def _stream_body(t_ref, x_ref, m_ref, s_ref):
    i = pl.program_id(0)

    @pl.when(i == 0)
    def _():
        m_ref[...] = jnp.full((B, 1), NEG_INF, jnp.float32)
        s_ref[...] = jnp.zeros((B, 1), jnp.float32)

    inv_t = 1.0 / t_ref[0]
    v = x_ref[...] * inv_t
    col = i * VC + lax.broadcasted_iota(jnp.int32, (B, VC), 1)
    vm = jnp.where(col < V, v, NEG_INF)
    m_old = m_ref[...]
    s_old = s_ref[...]
    m_new = jnp.maximum(m_old, jnp.max(vm, axis=1, keepdims=True))
    s_add = jnp.sum(jnp.exp(vm - m_new), axis=1, keepdims=True)
    m_ref[...] = m_new
    s_ref[...] = s_old * jnp.exp(m_old - m_new) + s_add


def _stream(t1, x):
    return pl.pallas_call(
        _stream_body,
        grid=(NB,),
        in_specs=[
            pl.BlockSpec(memory_space=pltpu.SMEM),
            pl.BlockSpec((B, VC), lambda i: (0, i)),
        ],
        out_specs=[
            pl.BlockSpec((B, 1), lambda i: (0, 0)),
            pl.BlockSpec((B, 1), lambda i: (0, 0)),
        ],
        out_shape=[jax.ShapeDtypeStruct((B, 1), jnp.float32)] * 2,
    )(t1, x)


def _combine_body(t_ref, b_ref, top_ref, xt_ref, m0_ref, s0_ref,
                  tok_ref, ids_ref, w_ref, out_ref):
    inv_t = 1.0 / t_ref[0]
    tv = top_ref[...] * inv_t
    m0 = m0_ref[...]
    s0 = s0_ref[...]
    temp = jnp.sum(tv * w_ref[...], axis=1, keepdims=True) + b_ref[0]
    temp = jnp.maximum(temp, 1e-6)
    s_minus = jnp.sum(jnp.exp(tv - m0), axis=1, keepdims=True)
    new_top = tv / temp
    m_r = jnp.maximum(m0, jnp.max(new_top, axis=1, keepdims=True))
    s_new = jnp.sum(jnp.exp(new_top - m_r), axis=1, keepdims=True)
    denom = jnp.maximum(s0 - s_minus, 0.0) * jnp.exp(m0 - m_r) + s_new
    in_top = jnp.any(ids_ref[...] == tok_ref[...], axis=1, keepdims=True)
    vt = xt_ref[...] * inv_t
    vt = jnp.where(in_top, vt / temp, vt)
    out_ref[...] = jnp.exp(vt - m_r) / denom


def _combine(t1, b1, top, xt, m0, s0, tok, ids, w):
    return pl.pallas_call(
        _combine_body,
        in_specs=[
            pl.BlockSpec(memory_space=pltpu.SMEM),
            pl.BlockSpec(memory_space=pltpu.SMEM),
            pl.BlockSpec((B, K), lambda: (0, 0)),
            pl.BlockSpec((B, 1), lambda: (0, 0)),
            pl.BlockSpec((B, 1), lambda: (0, 0)),
            pl.BlockSpec((B, 1), lambda: (0, 0)),
            pl.BlockSpec((B, 1), lambda: (0, 0)),
            pl.BlockSpec((1, K), lambda: (0, 0)),
            pl.BlockSpec((1, K), lambda: (0, 0)),
        ],
        out_specs=pl.BlockSpec((B, 1), lambda: (0, 0)),
        out_shape=jax.ShapeDtypeStruct((B, 1), jnp.float32),
    )(t1, b1, top, xt, m0, s0, tok, ids, w)


def kernel(x, tokens, top_token_ids, W, b, general_temp):
    xflat = x.reshape(B * V)
    t1 = general_temp.reshape(1)
    top3, xt = _sc_gather(xflat, top_token_ids, tokens)
    m0, s0 = _stream(t1, x)
    out = _combine(t1, b, top3.reshape(B, K), xt.reshape(B, 1), m0, s0,
                   tokens.reshape(B, 1), top_token_ids.reshape(1, K), W)
    return out.reshape(B)


# speedup vs baseline: 1.4422x; 1.4422x over previous
import functools

import jax
import jax.numpy as jnp
from jax import lax
from jax.experimental import pallas as pl
from jax.experimental.pallas import tpu as pltpu
from jax.experimental.pallas import tpu_sc as plsc

B = 128
V = 100000
K = 1024
NW = 32
IDS_PER_W = K // NW
TOK_W = 16
TOK_PER_W = B // TOK_W
VC = 2048
NB = (V + VC - 1) // VC
NEG_INF = float("-inf")


def _stream_body(t_ref, x_ref, m_ref, s_ref, xt_ref):
    i = pl.program_id(0)

    @pl.when(i == 0)
    def _():
        m_ref[...] = jnp.full((B, 1), NEG_INF, jnp.float32)
        s_ref[...] = jnp.zeros((B, 1), jnp.float32)

    inv_t = 1.0 / t_ref[0]
    v = x_ref[...] * inv_t
    xt_ref[...] = v.T
    col = i * VC + lax.broadcasted_iota(jnp.int32, (B, VC), 1)
    vm = jnp.where(col < V, v, NEG_INF)
    m_old = m_ref[...]
    s_old = s_ref[...]
    m_new = jnp.maximum(m_old, jnp.max(vm, axis=1, keepdims=True))
    s_add = jnp.sum(jnp.exp(vm - m_new), axis=1, keepdims=True)
    m_ref[...] = m_new
    s_ref[...] = s_old * jnp.exp(m_old - m_new) + s_add


def _stream(t1, x):
    return pl.pallas_call(
        _stream_body,
        grid=(NB,),
        in_specs=[
            pl.BlockSpec(memory_space=pltpu.SMEM),
            pl.BlockSpec((B, VC), lambda i: (0, i)),
        ],
        out_specs=[
            pl.BlockSpec((B, 1), lambda i: (0, 0)),
            pl.BlockSpec((B, 1), lambda i: (0, 0)),
            pl.BlockSpec((VC, B), lambda i: (i, 0)),
        ],
        out_shape=[
            jax.ShapeDtypeStruct((B, 1), jnp.float32),
            jax.ShapeDtypeStruct((B, 1), jnp.float32),
            jax.ShapeDtypeStruct((V, B), jnp.float32),
        ],
    )(t1, x)


def _sc_gather_body(xt_hbm, ids_hbm, tok_hbm, top_out, d_out,
                    ids_v, rows_v, tok_v, trows_v, sem, sem2):
    wid = lax.axis_index("c") * 16 + lax.axis_index("s")
    base = wid * IDS_PER_W
    pltpu.sync_copy(ids_hbm.at[pl.ds(base, IDS_PER_W)], ids_v)
    cp = pltpu.async_copy(xt_hbm.at[ids_v], rows_v, sem)

    @pl.when(wid < TOK_W)
    def _():
        tbase = wid * TOK_PER_W
        pltpu.sync_copy(tok_hbm.at[pl.ds(tbase, TOK_PER_W)], tok_v)
        pltpu.async_copy(xt_hbm.at[tok_v], trows_v, sem2).wait()
        pltpu.sync_copy(trows_v, d_out.at[pl.ds(tbase, TOK_PER_W)])

    cp.wait()
    pltpu.sync_copy(rows_v, top_out.at[pl.ds(base, IDS_PER_W)])


_sc_gather = functools.partial(
    pl.kernel,
    mesh=plsc.VectorSubcoreMesh(core_axis_name="c", subcore_axis_name="s"),
    out_type=[
        jax.ShapeDtypeStruct((K, B), jnp.float32),
        jax.ShapeDtypeStruct((B, B), jnp.float32),
    ],
    scratch_types=[
        pltpu.VMEM((IDS_PER_W,), jnp.int32),
        pltpu.VMEM((IDS_PER_W, B), jnp.float32),
        pltpu.VMEM((TOK_PER_W,), jnp.int32),
        pltpu.VMEM((TOK_PER_W, B), jnp.float32),
        pltpu.SemaphoreType.DMA,
        pltpu.SemaphoreType.DMA,
    ],
)(_sc_gather_body)


def _combine_body(b_ref, top_ref, d_ref, m0_ref, s0_ref,
                  tok_ref, ids_ref, w_ref, out_ref):
    tv = top_ref[...]
    m0 = m0_ref[...].T
    s0 = s0_ref[...].T
    temp = jnp.dot(w_ref[...], tv,
                   preferred_element_type=jnp.float32) + b_ref[0]
    temp = jnp.maximum(temp, 1e-6)
    s_minus = jnp.sum(jnp.exp(tv - m0), axis=0, keepdims=True)
    new_top = tv / temp
    m_r = jnp.maximum(m0, jnp.max(new_top, axis=0, keepdims=True))
    s_new = jnp.sum(jnp.exp(new_top - m_r), axis=0, keepdims=True)
    denom = jnp.maximum(s0 - s_minus, 0.0) * jnp.exp(m0 - m_r) + s_new
    in_top = jnp.any(ids_ref[...] == tok_ref[...], axis=0, keepdims=True)
    d = d_ref[...]
    eye = (lax.broadcasted_iota(jnp.int32, (B, B), 0)
           == lax.broadcasted_iota(jnp.int32, (B, B), 1))
    vt = jnp.sum(jnp.where(eye, d, 0.0), axis=0, keepdims=True)
    vt = jnp.where(in_top, vt / temp, vt)
    out_ref[...] = jnp.exp(vt - m_r) / denom


def _combine(b1, top, d, m0, s0, tok, ids, w):
    return pl.pallas_call(
        _combine_body,
        in_specs=[
            pl.BlockSpec(memory_space=pltpu.SMEM),
            pl.BlockSpec((K, B), lambda: (0, 0)),
            pl.BlockSpec((B, B), lambda: (0, 0)),
            pl.BlockSpec((B, 1), lambda: (0, 0)),
            pl.BlockSpec((B, 1), lambda: (0, 0)),
            pl.BlockSpec((1, B), lambda: (0, 0)),
            pl.BlockSpec((K, 1), lambda: (0, 0)),
            pl.BlockSpec((1, K), lambda: (0, 0)),
        ],
        out_specs=pl.BlockSpec((1, B), lambda: (0, 0)),
        out_shape=jax.ShapeDtypeStruct((1, B), jnp.float32),
    )(b1, top, d, m0, s0, tok, ids, w)


def kernel(x, tokens, top_token_ids, W, b, general_temp):
    t1 = general_temp.reshape(1)
    m0, s0, xt = _stream(t1, x)
    topT, dT = _sc_gather(xt, top_token_ids, tokens)
    out = _combine(b, topT, dT, m0, s0, tokens.reshape(1, B),
                   top_token_ids.reshape(K, 1), W)
    return out.reshape(B)


# speedup vs baseline: 1.4493x; 1.0050x over previous
import functools

import jax
import jax.numpy as jnp
from jax import lax
from jax.experimental import pallas as pl
from jax.experimental.pallas import tpu as pltpu
from jax.experimental.pallas import tpu_sc as plsc

B = 128
V = 100000
K = 1024
NW = 32
IDS_PER_W = K // NW
TOK_W = 16
TOK_PER_W = B // TOK_W
VC = 2048
NB = (V + VC - 1) // VC
NEG_INF = float("-inf")


def _stream_body(t_ref, x_ref, m_ref, s_ref, xt_ref):
    i = pl.program_id(0)

    @pl.when(i == 0)
    def _():
        m_ref[...] = jnp.full((B, 1), NEG_INF, jnp.float32)
        s_ref[...] = jnp.zeros((B, 1), jnp.float32)

    inv_t = 1.0 / t_ref[0]
    v = x_ref[...] * inv_t
    xt_ref[...] = v.T
    col = i * VC + lax.broadcasted_iota(jnp.int32, (B, VC), 1)
    vm = jnp.where(col < V, v, NEG_INF)
    m_old = m_ref[...]
    s_old = s_ref[...]
    m_new = jnp.maximum(m_old, jnp.max(vm, axis=1, keepdims=True))
    s_add = jnp.sum(jnp.exp(vm - m_new), axis=1, keepdims=True)
    m_ref[...] = m_new
    s_ref[...] = s_old * jnp.exp(m_old - m_new) + s_add


def _stream(t1, x):
    return pl.pallas_call(
        _stream_body,
        grid=(NB,),
        in_specs=[
            pl.BlockSpec(memory_space=pltpu.SMEM),
            pl.BlockSpec((B, VC), lambda i: (0, i)),
        ],
        out_specs=[
            pl.BlockSpec((B, 1), lambda i: (0, 0)),
            pl.BlockSpec((B, 1), lambda i: (0, 0)),
            pl.BlockSpec((VC, B), lambda i: (i, 0)),
        ],
        out_shape=[
            jax.ShapeDtypeStruct((B, 1), jnp.float32),
            jax.ShapeDtypeStruct((B, 1), jnp.float32),
            jax.ShapeDtypeStruct((V, B), jnp.float32),
        ],
    )(t1, x)


def _sc_gather_body(xt_hbm, ids_hbm, tok_hbm, top_out, d_out,
                    ids_v, rows_v, tok_v, trows_v, sem, sem2):
    wid = lax.axis_index("c") * 16 + lax.axis_index("s")
    base = wid * IDS_PER_W
    pltpu.sync_copy(ids_hbm.at[pl.ds(base, IDS_PER_W)], ids_v)
    cp = pltpu.async_copy(xt_hbm.at[ids_v], rows_v, sem)

    @pl.when(wid < TOK_W)
    def _():
        tbase = wid * TOK_PER_W
        pltpu.sync_copy(tok_hbm.at[pl.ds(tbase, TOK_PER_W)], tok_v)
        pltpu.async_copy(xt_hbm.at[tok_v], trows_v, sem2).wait()
        pltpu.sync_copy(trows_v, d_out.at[pl.ds(tbase, TOK_PER_W)])

    cp.wait()
    pltpu.sync_copy(rows_v, top_out.at[pl.ds(base, IDS_PER_W)])


_sc_gather = functools.partial(
    pl.kernel,
    mesh=plsc.VectorSubcoreMesh(core_axis_name="c", subcore_axis_name="s"),
    compiler_params=pltpu.CompilerParams(use_tc_tiling_on_sc=True),
    out_type=[
        jax.ShapeDtypeStruct((K, B), jnp.float32),
        jax.ShapeDtypeStruct((B, B), jnp.float32),
    ],
    scratch_types=[
        pltpu.VMEM((IDS_PER_W,), jnp.int32),
        pltpu.VMEM((IDS_PER_W, B), jnp.float32),
        pltpu.VMEM((TOK_PER_W,), jnp.int32),
        pltpu.VMEM((TOK_PER_W, B), jnp.float32),
        pltpu.SemaphoreType.DMA,
        pltpu.SemaphoreType.DMA,
    ],
)(_sc_gather_body)


def _combine_body(b_ref, top_ref, d_ref, m0_ref, s0_ref,
                  tok_ref, ids_ref, w_ref, out_ref):
    tv = top_ref[...]
    m0 = m0_ref[...].T
    s0 = s0_ref[...].T
    temp = jnp.dot(w_ref[...], tv,
                   preferred_element_type=jnp.float32) + b_ref[0]
    temp = jnp.maximum(temp, 1e-6)
    s_minus = jnp.sum(jnp.exp(tv - m0), axis=0, keepdims=True)
    new_top = tv / temp
    m_r = jnp.maximum(m0, jnp.max(new_top, axis=0, keepdims=True))
    s_new = jnp.sum(jnp.exp(new_top - m_r), axis=0, keepdims=True)
    denom = jnp.maximum(s0 - s_minus, 0.0) * jnp.exp(m0 - m_r) + s_new
    in_top = jnp.any(ids_ref[...] == tok_ref[...], axis=0, keepdims=True)
    d = d_ref[...]
    eye = (lax.broadcasted_iota(jnp.int32, (B, B), 0)
           == lax.broadcasted_iota(jnp.int32, (B, B), 1))
    vt = jnp.sum(jnp.where(eye, d, 0.0), axis=0, keepdims=True)
    vt = jnp.where(in_top, vt / temp, vt)
    out_ref[...] = jnp.exp(vt - m_r) / denom


def _combine(b1, top, d, m0, s0, tok, ids, w):
    return pl.pallas_call(
        _combine_body,
        in_specs=[
            pl.BlockSpec(memory_space=pltpu.SMEM),
            pl.BlockSpec((K, B), lambda: (0, 0)),
            pl.BlockSpec((B, B), lambda: (0, 0)),
            pl.BlockSpec((B, 1), lambda: (0, 0)),
            pl.BlockSpec((B, 1), lambda: (0, 0)),
            pl.BlockSpec((1, B), lambda: (0, 0)),
            pl.BlockSpec((K, 1), lambda: (0, 0)),
            pl.BlockSpec((1, K), lambda: (0, 0)),
        ],
        out_specs=pl.BlockSpec((1, B), lambda: (0, 0)),
        out_shape=jax.ShapeDtypeStruct((1, B), jnp.float32),
    )(b1, top, d, m0, s0, tok, ids, w)


def kernel(x, tokens, top_token_ids, W, b, general_temp):
    t1 = general_temp.reshape(1)
    m0, s0, xt = _stream(t1, x)
    topT, dT = _sc_gather(xt, top_token_ids, tokens)
    out = _combine(b, topT, dT, m0, s0, tokens.reshape(1, B),
                   top_token_ids.reshape(K, 1), W)
    return out.reshape(B)


# speedup vs baseline: 2.7596x; 1.9040x over previous
import functools

import jax
import jax.numpy as jnp
from jax import lax
from jax.experimental import pallas as pl
from jax.experimental.pallas import tpu as pltpu
from jax.experimental.pallas import tpu_sc as plsc

B = 128
V = 100000
K = 1024
NW = 32
IDS_PER_W = K // NW
TOK_W = 16
TOK_PER_W = B // TOK_W
VC = 2048
NB = (V + VC - 1) // VC
NEG_INF = float("-inf")


def _stream_body(t_ref, x_ref, m_ref, s_ref):
    i = pl.program_id(0)

    @pl.when(i == 0)
    def _():
        m_ref[...] = jnp.full((1, B), NEG_INF, jnp.float32)
        s_ref[...] = jnp.zeros((1, B), jnp.float32)

    inv_t = 1.0 / t_ref[0]
    v = x_ref[...] * inv_t
    row = i * VC + lax.broadcasted_iota(jnp.int32, (VC, B), 0)
    vm = jnp.where(row < V, v, NEG_INF)
    m_old = m_ref[...]
    s_old = s_ref[...]
    m_new = jnp.maximum(m_old, jnp.max(vm, axis=0, keepdims=True))
    s_add = jnp.sum(jnp.exp(vm - m_new), axis=0, keepdims=True)
    m_ref[...] = m_new
    s_ref[...] = s_old * jnp.exp(m_old - m_new) + s_add


def _stream(t1, xt):
    return pl.pallas_call(
        _stream_body,
        grid=(NB,),
        in_specs=[
            pl.BlockSpec(memory_space=pltpu.SMEM),
            pl.BlockSpec((VC, B), lambda i: (i, 0)),
        ],
        out_specs=[
            pl.BlockSpec((1, B), lambda i: (0, 0)),
            pl.BlockSpec((1, B), lambda i: (0, 0)),
        ],
        out_shape=[
            jax.ShapeDtypeStruct((1, B), jnp.float32),
            jax.ShapeDtypeStruct((1, B), jnp.float32),
        ],
    )(t1, xt)


def _sc_gather_body(xt_hbm, ids_hbm, tok_hbm, top_out, d_out,
                    ids_v, rows_v, tok_v, trows_v, sem, sem2):
    wid = lax.axis_index("c") * 16 + lax.axis_index("s")
    base = wid * IDS_PER_W
    pltpu.sync_copy(ids_hbm.at[pl.ds(base, IDS_PER_W)], ids_v)
    cp = pltpu.async_copy(xt_hbm.at[ids_v], rows_v, sem)

    @pl.when(wid < TOK_W)
    def _():
        tbase = wid * TOK_PER_W
        pltpu.sync_copy(tok_hbm.at[pl.ds(tbase, TOK_PER_W)], tok_v)
        pltpu.async_copy(xt_hbm.at[tok_v], trows_v, sem2).wait()
        pltpu.sync_copy(trows_v, d_out.at[pl.ds(tbase, TOK_PER_W)])

    cp.wait()
    pltpu.sync_copy(rows_v, top_out.at[pl.ds(base, IDS_PER_W)])


_sc_gather = functools.partial(
    pl.kernel,
    mesh=plsc.VectorSubcoreMesh(core_axis_name="c", subcore_axis_name="s"),
    out_type=[
        jax.ShapeDtypeStruct((K, B), jnp.float32),
        jax.ShapeDtypeStruct((B, B), jnp.float32),
    ],
    scratch_types=[
        pltpu.VMEM((IDS_PER_W,), jnp.int32),
        pltpu.VMEM((IDS_PER_W, B), jnp.float32),
        pltpu.VMEM((TOK_PER_W,), jnp.int32),
        pltpu.VMEM((TOK_PER_W, B), jnp.float32),
        pltpu.SemaphoreType.DMA,
        pltpu.SemaphoreType.DMA,
    ],
)(_sc_gather_body)


def _combine_body(t_ref, b_ref, top_ref, d_ref, m0_ref, s0_ref,
                  tok_ref, ids_ref, w_ref, out_ref):
    inv_t = 1.0 / t_ref[0]
    tv = top_ref[...] * inv_t
    m0 = m0_ref[...]
    s0 = s0_ref[...]
    temp = jnp.dot(w_ref[...], tv,
                   preferred_element_type=jnp.float32) + b_ref[0]
    temp = jnp.maximum(temp, 1e-6)
    s_minus = jnp.sum(jnp.exp(tv - m0), axis=0, keepdims=True)
    new_top = tv / temp
    m_r = jnp.maximum(m0, jnp.max(new_top, axis=0, keepdims=True))
    s_new = jnp.sum(jnp.exp(new_top - m_r), axis=0, keepdims=True)
    denom = jnp.maximum(s0 - s_minus, 0.0) * jnp.exp(m0 - m_r) + s_new
    in_top = jnp.any(ids_ref[...] == tok_ref[...], axis=0, keepdims=True)
    d = d_ref[...]
    eye = (lax.broadcasted_iota(jnp.int32, (B, B), 0)
           == lax.broadcasted_iota(jnp.int32, (B, B), 1))
    vt = jnp.sum(jnp.where(eye, d, 0.0), axis=0, keepdims=True) * inv_t
    vt = jnp.where(in_top, vt / temp, vt)
    out_ref[...] = jnp.exp(vt - m_r) / denom


def _combine(t1, b1, top, d, m0, s0, tok, ids, w):
    return pl.pallas_call(
        _combine_body,
        in_specs=[
            pl.BlockSpec(memory_space=pltpu.SMEM),
            pl.BlockSpec(memory_space=pltpu.SMEM),
            pl.BlockSpec((K, B), lambda: (0, 0)),
            pl.BlockSpec((B, B), lambda: (0, 0)),
            pl.BlockSpec((1, B), lambda: (0, 0)),
            pl.BlockSpec((1, B), lambda: (0, 0)),
            pl.BlockSpec((1, B), lambda: (0, 0)),
            pl.BlockSpec((K, 1), lambda: (0, 0)),
            pl.BlockSpec((1, K), lambda: (0, 0)),
        ],
        out_specs=pl.BlockSpec((1, B), lambda: (0, 0)),
        out_shape=jax.ShapeDtypeStruct((1, B), jnp.float32),
    )(t1, b1, top, d, m0, s0, tok, ids, w)


def kernel(x, tokens, top_token_ids, W, b, general_temp):
    t1 = general_temp.reshape(1)
    xt = x.T
    topT, dT = _sc_gather(xt, top_token_ids, tokens)
    m0, s0 = _stream(t1, xt)
    out = _combine(t1, b, topT, dT, m0, s0, tokens.reshape(1, B),
                   top_token_ids.reshape(K, 1), W)
    return out.reshape(B)


# speedup vs baseline: 3.3179x; 1.2023x over previous
import functools

import jax
import jax.numpy as jnp
from jax import lax
from jax.experimental import pallas as pl
from jax.experimental.pallas import tpu as pltpu
from jax.experimental.pallas import tpu_sc as plsc

B = 128
V = 100000
K = 1024
NW = 32
IDS_PER_W = K // NW
TOK_W = 16
TOK_PER_W = B // TOK_W
VC = 4096
NB = (V + VC - 1) // VC
NEG_INF = float("-inf")


def _stream_body(t_ref, x_ref, m_ref, s_ref):
    i = pl.program_id(0)

    @pl.when(i == 0)
    def _():
        m_ref[...] = jnp.full((1, B), NEG_INF, jnp.float32)
        s_ref[...] = jnp.zeros((1, B), jnp.float32)

    inv_t = 1.0 / t_ref[0]
    v = x_ref[...] * inv_t
    row = i * VC + lax.broadcasted_iota(jnp.int32, (VC, B), 0)
    vm = jnp.where(row < V, v, NEG_INF)
    m_old = m_ref[...]
    s_old = s_ref[...]
    m_new = jnp.maximum(m_old, jnp.max(vm, axis=0, keepdims=True))
    s_add = jnp.sum(jnp.exp(vm - m_new), axis=0, keepdims=True)
    m_ref[...] = m_new
    s_ref[...] = s_old * jnp.exp(m_old - m_new) + s_add


def _stream(t1, xt):
    return pl.pallas_call(
        _stream_body,
        grid=(NB,),
        in_specs=[
            pl.BlockSpec(memory_space=pltpu.SMEM),
            pl.BlockSpec((VC, B), lambda i: (i, 0)),
        ],
        out_specs=[
            pl.BlockSpec((1, B), lambda i: (0, 0)),
            pl.BlockSpec((1, B), lambda i: (0, 0)),
        ],
        out_shape=[
            jax.ShapeDtypeStruct((1, B), jnp.float32),
            jax.ShapeDtypeStruct((1, B), jnp.float32),
        ],
    )(t1, xt)


def _sc_gather_body(xt_hbm, ids_hbm, tok_hbm, top_out, d_out,
                    ids_v, rows_v, tok_v, trows_v, sem, sem2):
    wid = lax.axis_index("c") * 16 + lax.axis_index("s")
    base = wid * IDS_PER_W
    pltpu.sync_copy(ids_hbm.at[pl.ds(base, IDS_PER_W)], ids_v)
    cp = pltpu.async_copy(xt_hbm.at[ids_v], rows_v, sem)

    @pl.when(wid < TOK_W)
    def _():
        tbase = wid * TOK_PER_W
        pltpu.sync_copy(tok_hbm.at[pl.ds(tbase, TOK_PER_W)], tok_v)
        pltpu.async_copy(xt_hbm.at[tok_v], trows_v, sem2).wait()
        pltpu.sync_copy(trows_v, d_out.at[pl.ds(tbase, TOK_PER_W)])

    cp.wait()
    pltpu.sync_copy(rows_v, top_out.at[pl.ds(base, IDS_PER_W)])


_sc_gather = functools.partial(
    pl.kernel,
    mesh=plsc.VectorSubcoreMesh(core_axis_name="c", subcore_axis_name="s"),
    out_type=[
        jax.ShapeDtypeStruct((K, B), jnp.float32),
        jax.ShapeDtypeStruct((B, B), jnp.float32),
    ],
    scratch_types=[
        pltpu.VMEM((IDS_PER_W,), jnp.int32),
        pltpu.VMEM((IDS_PER_W, B), jnp.float32),
        pltpu.VMEM((TOK_PER_W,), jnp.int32),
        pltpu.VMEM((TOK_PER_W, B), jnp.float32),
        pltpu.SemaphoreType.DMA,
        pltpu.SemaphoreType.DMA,
    ],
)(_sc_gather_body)


def _combine_body(t_ref, b_ref, top_ref, d_ref, m0_ref, s0_ref,
                  tok_ref, ids_ref, w_ref, out_ref):
    inv_t = 1.0 / t_ref[0]
    tv = top_ref[...] * inv_t
    m0 = m0_ref[...]
    s0 = s0_ref[...]
    temp = jnp.dot(w_ref[...], tv,
                   preferred_element_type=jnp.float32) + b_ref[0]
    temp = jnp.maximum(temp, 1e-6)
    s_minus = jnp.sum(jnp.exp(tv - m0), axis=0, keepdims=True)
    new_top = tv / temp
    m_r = jnp.maximum(m0, jnp.max(new_top, axis=0, keepdims=True))
    s_new = jnp.sum(jnp.exp(new_top - m_r), axis=0, keepdims=True)
    denom = jnp.maximum(s0 - s_minus, 0.0) * jnp.exp(m0 - m_r) + s_new
    in_top = jnp.any(ids_ref[...] == tok_ref[...], axis=0, keepdims=True)
    d = d_ref[...]
    eye = (lax.broadcasted_iota(jnp.int32, (B, B), 0)
           == lax.broadcasted_iota(jnp.int32, (B, B), 1))
    vt = jnp.sum(jnp.where(eye, d, 0.0), axis=0, keepdims=True) * inv_t
    vt = jnp.where(in_top, vt / temp, vt)
    out_ref[...] = jnp.exp(vt - m_r) / denom


def _combine(t1, b1, top, d, m0, s0, tok, ids, w):
    return pl.pallas_call(
        _combine_body,
        in_specs=[
            pl.BlockSpec(memory_space=pltpu.SMEM),
            pl.BlockSpec(memory_space=pltpu.SMEM),
            pl.BlockSpec((K, B), lambda: (0, 0)),
            pl.BlockSpec((B, B), lambda: (0, 0)),
            pl.BlockSpec((1, B), lambda: (0, 0)),
            pl.BlockSpec((1, B), lambda: (0, 0)),
            pl.BlockSpec((1, B), lambda: (0, 0)),
            pl.BlockSpec((K, 1), lambda: (0, 0)),
            pl.BlockSpec((1, K), lambda: (0, 0)),
        ],
        out_specs=pl.BlockSpec((1, B), lambda: (0, 0)),
        out_shape=jax.ShapeDtypeStruct((1, B), jnp.float32),
    )(t1, b1, top, d, m0, s0, tok, ids, w)


def kernel(x, tokens, top_token_ids, W, b, general_temp):
    t1 = general_temp.reshape(1)
    xt = x.T
    topT, dT = _sc_gather(xt, top_token_ids, tokens)
    m0, s0 = _stream(t1, xt)
    out = _combine(t1, b, topT, dT, m0, s0, tokens.reshape(1, B),
                   top_token_ids.reshape(K, 1), W)
    return out.reshape(B)


# speedup vs baseline: 3.5559x; 1.0717x over previous
import functools

import jax
import jax.numpy as jnp
from jax import lax
from jax.experimental import pallas as pl
from jax.experimental.pallas import tpu as pltpu
from jax.experimental.pallas import tpu_sc as plsc

B = 128
V = 100000
K = 1024
NW = 32
IDS_PER_W = K // NW
TOK_W = 16
TOK_PER_W = B // TOK_W
VC = 8192
NB = (V + VC - 1) // VC
NEG_INF = float("-inf")


def _stream_body(t_ref, x_ref, m_ref, s_ref):
    i = pl.program_id(0)

    @pl.when(i == 0)
    def _():
        m_ref[...] = jnp.full((1, B), NEG_INF, jnp.float32)
        s_ref[...] = jnp.zeros((1, B), jnp.float32)

    inv_t = 1.0 / t_ref[0]
    v = x_ref[...] * inv_t
    row = i * VC + lax.broadcasted_iota(jnp.int32, (VC, B), 0)
    vm = jnp.where(row < V, v, NEG_INF)
    m_old = m_ref[...]
    s_old = s_ref[...]
    m_new = jnp.maximum(m_old, jnp.max(vm, axis=0, keepdims=True))
    s_add = jnp.sum(jnp.exp(vm - m_new), axis=0, keepdims=True)
    m_ref[...] = m_new
    s_ref[...] = s_old * jnp.exp(m_old - m_new) + s_add


def _stream(t1, xt):
    return pl.pallas_call(
        _stream_body,
        grid=(NB,),
        in_specs=[
            pl.BlockSpec(memory_space=pltpu.SMEM),
            pl.BlockSpec((VC, B), lambda i: (i, 0)),
        ],
        out_specs=[
            pl.BlockSpec((1, B), lambda i: (0, 0)),
            pl.BlockSpec((1, B), lambda i: (0, 0)),
        ],
        out_shape=[
            jax.ShapeDtypeStruct((1, B), jnp.float32),
            jax.ShapeDtypeStruct((1, B), jnp.float32),
        ],
    )(t1, xt)


def _sc_gather_body(xt_hbm, ids_hbm, tok_hbm, top_out, d_out,
                    ids_v, rows_v, tok_v, trows_v, sem, sem2):
    wid = lax.axis_index("c") * 16 + lax.axis_index("s")
    base = wid * IDS_PER_W
    pltpu.sync_copy(ids_hbm.at[pl.ds(base, IDS_PER_W)], ids_v)
    cp = pltpu.async_copy(xt_hbm.at[ids_v], rows_v, sem)

    @pl.when(wid < TOK_W)
    def _():
        tbase = wid * TOK_PER_W
        pltpu.sync_copy(tok_hbm.at[pl.ds(tbase, TOK_PER_W)], tok_v)
        pltpu.async_copy(xt_hbm.at[tok_v], trows_v, sem2).wait()
        pltpu.sync_copy(trows_v, d_out.at[pl.ds(tbase, TOK_PER_W)])

    cp.wait()
    pltpu.sync_copy(rows_v, top_out.at[pl.ds(base, IDS_PER_W)])


_sc_gather = functools.partial(
    pl.kernel,
    mesh=plsc.VectorSubcoreMesh(core_axis_name="c", subcore_axis_name="s"),
    out_type=[
        jax.ShapeDtypeStruct((K, B), jnp.float32),
        jax.ShapeDtypeStruct((B, B), jnp.float32),
    ],
    scratch_types=[
        pltpu.VMEM((IDS_PER_W,), jnp.int32),
        pltpu.VMEM((IDS_PER_W, B), jnp.float32),
        pltpu.VMEM((TOK_PER_W,), jnp.int32),
        pltpu.VMEM((TOK_PER_W, B), jnp.float32),
        pltpu.SemaphoreType.DMA,
        pltpu.SemaphoreType.DMA,
    ],
)(_sc_gather_body)


def _combine_body(t_ref, b_ref, top_ref, d_ref, m0_ref, s0_ref,
                  tok_ref, ids_ref, w_ref, out_ref):
    inv_t = 1.0 / t_ref[0]
    tv = top_ref[...] * inv_t
    m0 = m0_ref[...]
    s0 = s0_ref[...]
    temp = jnp.dot(w_ref[...], tv,
                   preferred_element_type=jnp.float32) + b_ref[0]
    temp = jnp.maximum(temp, 1e-6)
    s_minus = jnp.sum(jnp.exp(tv - m0), axis=0, keepdims=True)
    new_top = tv / temp
    m_r = jnp.maximum(m0, jnp.max(new_top, axis=0, keepdims=True))
    s_new = jnp.sum(jnp.exp(new_top - m_r), axis=0, keepdims=True)
    denom = jnp.maximum(s0 - s_minus, 0.0) * jnp.exp(m0 - m_r) + s_new
    in_top = jnp.any(ids_ref[...] == tok_ref[...], axis=0, keepdims=True)
    d = d_ref[...]
    eye = (lax.broadcasted_iota(jnp.int32, (B, B), 0)
           == lax.broadcasted_iota(jnp.int32, (B, B), 1))
    vt = jnp.sum(jnp.where(eye, d, 0.0), axis=0, keepdims=True) * inv_t
    vt = jnp.where(in_top, vt / temp, vt)
    out_ref[...] = jnp.exp(vt - m_r) / denom


def _combine(t1, b1, top, d, m0, s0, tok, ids, w):
    return pl.pallas_call(
        _combine_body,
        in_specs=[
            pl.BlockSpec(memory_space=pltpu.SMEM),
            pl.BlockSpec(memory_space=pltpu.SMEM),
            pl.BlockSpec((K, B), lambda: (0, 0)),
            pl.BlockSpec((B, B), lambda: (0, 0)),
            pl.BlockSpec((1, B), lambda: (0, 0)),
            pl.BlockSpec((1, B), lambda: (0, 0)),
            pl.BlockSpec((1, B), lambda: (0, 0)),
            pl.BlockSpec((K, 1), lambda: (0, 0)),
            pl.BlockSpec((1, K), lambda: (0, 0)),
        ],
        out_specs=pl.BlockSpec((1, B), lambda: (0, 0)),
        out_shape=jax.ShapeDtypeStruct((1, B), jnp.float32),
    )(t1, b1, top, d, m0, s0, tok, ids, w)


def kernel(x, tokens, top_token_ids, W, b, general_temp):
    t1 = general_temp.reshape(1)
    xt = x.T
    topT, dT = _sc_gather(xt, top_token_ids, tokens)
    m0, s0 = _stream(t1, xt)
    out = _combine(t1, b, topT, dT, m0, s0, tokens.reshape(1, B),
                   top_token_ids.reshape(K, 1), W)
    return out.reshape(B)
